# Initial kernel scaffold; baseline (speedup 1.0000x reference)
import jax
import jax.numpy as jnp
from jax.experimental import pallas as pl


def kernel(h, x, edges, edge_attr, params):
    raise NotImplementedError("write your pallas kernel here")


# speedup vs baseline: 1.7891x; 1.7891x over previous
import functools

import jax
import jax.numpy as jnp
from jax import lax
from jax.experimental import pallas as pl
from jax.experimental.pallas import tpu as pltpu
from jax.experimental.pallas import tpu_sc as plsc

F32 = jnp.float32
I32 = jnp.int32

N = 10000
E = 320000
H = 128
DE = 16
REC = H + 16
EPSV = 1e-8

NC, NS = 2, 16
NW = NC * NS
N_PAD = 10016
E_PAD = 327680
PER_W = E_PAD // NW
CH = 128
ITERS = PER_W // CH
RPT = N_PAD // NS

BE = 2048
BN = 2504


def _silu(v):
    return v * jax.nn.sigmoid(v)


def _init_body(h_ref, cp_ref, wemb_ref, bemb_ref, w1r_ref, w1c_ref,
               h0_ref, trow_ref, tcol_ref):
    h0 = jnp.dot(h_ref[...], wemb_ref[...], preferred_element_type=F32) + bemb_ref[...]
    h0_ref[...] = h0
    cp = cp_ref[...]
    pr = jnp.dot(h0, w1r_ref[...], preferred_element_type=F32)
    pc = jnp.dot(h0, w1c_ref[...], preferred_element_type=F32)
    trow_ref[...] = jnp.concatenate([pr, cp], axis=1)
    tcol_ref[...] = jnp.concatenate([pc, -cp], axis=1)


def _edge_body(s_ref, ea_ref, w1ea_ref, b1_ref, wr_ref, w2_ref, b2_ref,
               wc1_ref, bc1_ref, wc2_ref, out_ref):
    s = s_ref[...]
    sh = s[:, :H]
    diff = s[:, H:]
    radial = jnp.sum(diff * diff, axis=1, keepdims=True)
    norm = jnp.sqrt(radial) + EPSV
    unit = diff / norm
    e1 = sh + radial * wr_ref[...] + b1_ref[...]
    e1 = e1 + jnp.dot(ea_ref[...], w1ea_ref[...], preferred_element_type=F32)
    m = _silu(e1)
    m = _silu(jnp.dot(m, w2_ref[...], preferred_element_type=F32) + b2_ref[...])
    ch = _silu(jnp.dot(m, wc1_ref[...], preferred_element_type=F32) + bc1_ref[...])
    c = jnp.sum(ch * wc2_ref[...], axis=1, keepdims=True)
    trans = unit * c
    lane = lax.broadcasted_iota(I32, trans.shape, 1)
    trans = jnp.where(lane == 3, 1.0, trans)
    out_ref[...] = jnp.concatenate([m, trans], axis=1)


def _node_body(h_ref, cp_ref, p0_ref, p1_ref, wn1h_ref, wn1m_ref, bn1_ref,
               wn2_ref, bn2_ref, w1r_ref, w1c_ref, h1_ref, cp1_ref,
               *rest, last):
    p0 = p0_ref[...]
    p1 = p1_ref[...]
    magg = p0[:, :H] + p1[:, :H]
    tail = p0[:, H:] + p1[:, H:]
    cnt = jnp.maximum(tail[:, 3:4], 1.0)
    lane = lax.broadcasted_iota(I32, tail.shape, 1)
    aggt = jnp.where(lane < 3, tail, 0.0)
    cp1 = cp_ref[...] + aggt / cnt
    cp1_ref[...] = cp1
    h = h_ref[...]
    o = _silu(jnp.dot(h, wn1h_ref[...], preferred_element_type=F32)
              + jnp.dot(magg, wn1m_ref[...], preferred_element_type=F32)
              + bn1_ref[...])
    o = jnp.dot(o, wn2_ref[...], preferred_element_type=F32) + bn2_ref[...]
    h1 = h + o
    if last:
        h1_ref[...] = jnp.dot(h1, w1r_ref[...], preferred_element_type=F32) + w1c_ref[...]
    else:
        trow_ref, tcol_ref = rest
        h1_ref[...] = h1
        pr = jnp.dot(h1, w1r_ref[...], preferred_element_type=F32)
        pc = jnp.dot(h1, w1c_ref[...], preferred_element_type=F32)
        trow_ref[...] = jnp.concatenate([pr, cp1], axis=1)
        tcol_ref[...] = jnp.concatenate([pc, -cp1], axis=1)


def _wspec(r, c):
    return pl.BlockSpec((r, c), lambda i: (0, 0))


def _tc_init(h_pad, cp, wemb, bemb, w1r, w1c):
    grid = (N_PAD // BN,)
    return pl.pallas_call(
        _init_body,
        grid=grid,
        in_specs=[
            pl.BlockSpec((BN, H), lambda i: (i, 0)),
            pl.BlockSpec((BN, 16), lambda i: (i, 0)),
            _wspec(H, H), _wspec(1, H), _wspec(H, H), _wspec(H, H),
        ],
        out_specs=[
            pl.BlockSpec((BN, H), lambda i: (i, 0)),
            pl.BlockSpec((BN, REC), lambda i: (i, 0)),
            pl.BlockSpec((BN, REC), lambda i: (i, 0)),
        ],
        out_shape=[
            jax.ShapeDtypeStruct((N_PAD, H), F32),
            jax.ShapeDtypeStruct((N_PAD, REC), F32),
            jax.ShapeDtypeStruct((N_PAD, REC), F32),
        ],
    )(h_pad, cp, wemb, bemb, w1r, w1c)


def _tc_edge(s, ea, w1ea, b1, wr, w2, b2, wc1, bc1, wc2):
    grid = (E_PAD // BE,)
    return pl.pallas_call(
        _edge_body,
        grid=grid,
        in_specs=[
            pl.BlockSpec((BE, REC), lambda i: (i, 0)),
            pl.BlockSpec((BE, DE), lambda i: (i, 0)),
            _wspec(DE, H), _wspec(1, H), _wspec(1, H),
            _wspec(H, H), _wspec(1, H),
            _wspec(H, H), _wspec(1, H), _wspec(1, H),
        ],
        out_specs=pl.BlockSpec((BE, REC), lambda i: (i, 0)),
        out_shape=jax.ShapeDtypeStruct((E_PAD, REC), F32),
    )(s, ea, w1ea, b1, wr, w2, b2, wc1, bc1, wc2)


def _tc_node(h, cp, p0, p1, wn1h, wn1m, bn1, wn2, bn2, w1r, w1c, last):
    grid = (N_PAD // BN,)
    out_specs = [
        pl.BlockSpec((BN, H), lambda i: (i, 0)),
        pl.BlockSpec((BN, 16), lambda i: (i, 0)),
    ]
    out_shape = [
        jax.ShapeDtypeStruct((N_PAD, H), F32),
        jax.ShapeDtypeStruct((N_PAD, 16), F32),
    ]
    if not last:
        out_specs += [pl.BlockSpec((BN, REC), lambda i: (i, 0))] * 2
        out_shape += [jax.ShapeDtypeStruct((N_PAD, REC), F32)] * 2
    return pl.pallas_call(
        functools.partial(_node_body, last=last),
        grid=grid,
        in_specs=[
            pl.BlockSpec((BN, H), lambda i: (i, 0)),
            pl.BlockSpec((BN, 16), lambda i: (i, 0)),
            pl.BlockSpec((BN, REC), lambda i: (i, 0)),
            pl.BlockSpec((BN, REC), lambda i: (i, 0)),
            _wspec(H, H), _wspec(H, H), _wspec(1, H),
            _wspec(H, H), _wspec(1, H), _wspec(H, H),
            _wspec(H, H) if not last else _wspec(1, H),
        ],
        out_specs=out_specs,
        out_shape=out_shape,
    )(h, cp, p0, p1, wn1h, wn1m, bn1, wn2, bn2, w1r, w1c)


def _sc_gather_body(trow, tcol, rowg, colg, out, idx_r, idx_c, buf, sem):
    cid = lax.axis_index("c")
    sid = lax.axis_index("s")
    wid = sid * NC + cid
    base = wid * PER_W

    def body(j, carry):
        off = base + j * CH
        pltpu.sync_copy(rowg.at[pl.ds(off, CH)], idx_r)
        pltpu.sync_copy(colg.at[pl.ds(off, CH)], idx_c)
        pltpu.async_copy(trow.at[idx_r], buf, sem).wait()
        pltpu.async_copy(tcol.at[idx_c], buf, sem, add=True).wait()
        pltpu.sync_copy(buf, out.at[pl.ds(off, CH)])
        return carry

    lax.fori_loop(0, ITERS, body, 0)


def _sc_gather(trow, tcol, rowg, colg):
    mesh = plsc.VectorSubcoreMesh(core_axis_name="c", subcore_axis_name="s")
    f = functools.partial(
        pl.kernel,
        out_type=jax.ShapeDtypeStruct((E_PAD, REC), F32),
        mesh=mesh,
        compiler_params=pltpu.CompilerParams(use_tc_tiling_on_sc=False),
        scratch_types=[
            pltpu.VMEM((CH,), I32),
            pltpu.VMEM((CH,), I32),
            pltpu.VMEM((CH, REC), F32),
            pltpu.SemaphoreType.DMA,
        ],
    )(_sc_gather_body)
    return f(trow, tcol, rowg, colg)


def _sc_scatter_body(edat, rows, zeros, out, idx_v, buf, acc):
    cid = lax.axis_index("c")
    sid = lax.axis_index("s")
    wid = sid * NC + cid
    base = wid * PER_W

    pltpu.sync_copy(zeros.at[pl.ds(sid * RPT, RPT)], acc.at[pl.ds(sid * RPT, RPT)])
    plsc.subcore_barrier()

    def body(j, carry):
        off = base + j * CH
        pltpu.sync_copy(rows.at[pl.ds(off, CH)], idx_v)
        pltpu.sync_copy(edat.at[pl.ds(off, CH)], buf)
        pltpu.sync_copy(buf, acc.at[idx_v], add=True)
        return carry

    lax.fori_loop(0, ITERS, body, 0)
    plsc.subcore_barrier()
    pltpu.sync_copy(acc.at[pl.ds(sid * RPT, RPT)], out.at[cid].at[pl.ds(sid * RPT, RPT)])


def _sc_scatter(edat, rows, zeros):
    mesh = plsc.VectorSubcoreMesh(core_axis_name="c", subcore_axis_name="s")
    f = functools.partial(
        pl.kernel,
        out_type=jax.ShapeDtypeStruct((NC, N_PAD, REC), F32),
        mesh=mesh,
        compiler_params=pltpu.CompilerParams(use_tc_tiling_on_sc=False),
        scratch_types=[
            pltpu.VMEM((CH,), I32),
            pltpu.VMEM((CH, REC), F32),
            pltpu.VMEM_SHARED((N_PAD, REC), F32),
        ],
    )(_sc_scatter_body)
    return f(edat, rows, zeros)


def kernel(h, x, edges, edge_attr, params):
    row = edges[0]
    col = edges[1]

    h_pad = jnp.pad(h, ((0, N_PAD - N), (0, 0)))
    cp = jnp.pad(x, ((0, N_PAD - N), (0, 16 - 3)))
    ea = jnp.pad(edge_attr, ((0, E_PAD - E), (0, 0)))
    rowg = jnp.pad(row, (0, E_PAD - E))
    colg = jnp.pad(col, (0, E_PAD - E))
    rows_sc = jnp.pad(row, (0, E_PAD - E), constant_values=N)
    zeros = jnp.zeros((N_PAD, REC), F32)

    def lw(i):
        p = params["layers"][i]
        w1 = p["e1"]["W"]
        return {
            "w1r": w1[:, :H].T, "w1c": w1[:, H:2 * H].T,
            "wr": w1[:, 2 * H].reshape(1, H), "w1ea": w1[:, 2 * H + 1:].T,
            "b1": p["e1"]["b"].reshape(1, H),
            "w2": p["e2"]["W"].T, "b2": p["e2"]["b"].reshape(1, H),
            "wc1": p["c1"]["W"].T, "bc1": p["c1"]["b"].reshape(1, H),
            "wc2": p["c2"]["W"].reshape(1, H),
            "wn1h": p["n1"]["W"][:, :H].T, "wn1m": p["n1"]["W"][:, H:].T,
            "bn1": p["n1"]["b"].reshape(1, H),
            "wn2": p["n2"]["W"].T, "bn2": p["n2"]["b"].reshape(1, H),
        }

    lws = [lw(i) for i in range(3)]
    wemb_in = params["emb_in"]["W"].T
    bemb_in = params["emb_in"]["b"].reshape(1, H)
    wemb_out = params["emb_out"]["W"].T
    bemb_out = params["emb_out"]["b"].reshape(1, H)

    hcur, trow, tcol = _tc_init(h_pad, cp, wemb_in, bemb_in,
                                lws[0]["w1r"], lws[0]["w1c"])
    for i in range(3):
        w = lws[i]
        s = _sc_gather(trow, tcol, rowg, colg)
        edat = _tc_edge(s, ea, w["w1ea"], w["b1"], w["wr"], w["w2"], w["b2"],
                        w["wc1"], w["bc1"], w["wc2"])
        parts = _sc_scatter(edat, rows_sc, zeros)
        last = i == 2
        if last:
            nw1r, nw1c = wemb_out, bemb_out
        else:
            nw1r, nw1c = lws[i + 1]["w1r"], lws[i + 1]["w1c"]
        res = _tc_node(hcur, cp, parts[0], parts[1],
                       w["wn1h"], w["wn1m"], w["bn1"], w["wn2"], w["bn2"],
                       nw1r, nw1c, last)
        if last:
            hcur, cp = res
        else:
            hcur, cp, trow, tcol = res

    return hcur[:N], cp[:N, :3]


# speedup vs baseline: 2.2324x; 1.2478x over previous
import functools

import jax
import jax.numpy as jnp
from jax import lax
from jax.experimental import pallas as pl
from jax.experimental.pallas import tpu as pltpu
from jax.experimental.pallas import tpu_sc as plsc

F32 = jnp.float32
I32 = jnp.int32

N = 10000
E = 320000
H = 128
DE = 16
REC = H + 16
EPSV = 1e-8

NC, NS = 2, 16
NW = NC * NS
N_PAD = 10016
E_PAD = 327680
PER_W = E_PAD // NW
CH = 128
ITERS = PER_W // CH
CH_S = 64
ITERS_S = PER_W // CH_S
RPT = N_PAD // NS

BE = 2048
BN = 2504


def _silu(v):
    return v * jax.nn.sigmoid(v)


def _init_body(h_ref, cp_ref, wemb_ref, bemb_ref, w1r_ref, w1c_ref,
               h0_ref, trow_ref, tcol_ref):
    h0 = jnp.dot(h_ref[...], wemb_ref[...], preferred_element_type=F32) + bemb_ref[...]
    h0_ref[...] = h0
    cp = cp_ref[...]
    pr = jnp.dot(h0, w1r_ref[...], preferred_element_type=F32)
    pc = jnp.dot(h0, w1c_ref[...], preferred_element_type=F32)
    trow_ref[...] = jnp.concatenate([pr, cp], axis=1)
    tcol_ref[...] = jnp.concatenate([pc, -cp], axis=1)


def _edge_body(s_ref, ea_ref, w1ea_ref, b1_ref, wr_ref, w2_ref, b2_ref,
               wc1_ref, bc1_ref, wc2_ref, out_ref):
    s = s_ref[...]
    sh = s[:, :H]
    diff = s[:, H:]
    radial = jnp.sum(diff * diff, axis=1, keepdims=True)
    norm = jnp.sqrt(radial) + EPSV
    unit = diff / norm
    e1 = sh + radial * wr_ref[...] + b1_ref[...]
    e1 = e1 + jnp.dot(ea_ref[...], w1ea_ref[...], preferred_element_type=F32)
    m = _silu(e1)
    m = _silu(jnp.dot(m, w2_ref[...], preferred_element_type=F32) + b2_ref[...])
    ch = _silu(jnp.dot(m, wc1_ref[...], preferred_element_type=F32) + bc1_ref[...])
    c = jnp.sum(ch * wc2_ref[...], axis=1, keepdims=True)
    trans = unit * c
    lane = lax.broadcasted_iota(I32, trans.shape, 1)
    trans = jnp.where(lane == 3, 1.0, trans)
    out_ref[...] = jnp.concatenate([m, trans], axis=1)


def _node_body(h_ref, cp_ref, p0_ref, p1_ref, wn1h_ref, wn1m_ref, bn1_ref,
               wn2_ref, bn2_ref, w1r_ref, w1c_ref, h1_ref, cp1_ref,
               *rest, last):
    p0 = p0_ref[...]
    p1 = p1_ref[...]
    magg = p0[:, :H] + p1[:, :H]
    tail = p0[:, H:] + p1[:, H:]
    cnt = jnp.maximum(tail[:, 3:4], 1.0)
    lane = lax.broadcasted_iota(I32, tail.shape, 1)
    aggt = jnp.where(lane < 3, tail, 0.0)
    cp1 = cp_ref[...] + aggt / cnt
    cp1_ref[...] = cp1
    h = h_ref[...]
    o = _silu(jnp.dot(h, wn1h_ref[...], preferred_element_type=F32)
              + jnp.dot(magg, wn1m_ref[...], preferred_element_type=F32)
              + bn1_ref[...])
    o = jnp.dot(o, wn2_ref[...], preferred_element_type=F32) + bn2_ref[...]
    h1 = h + o
    if last:
        h1_ref[...] = jnp.dot(h1, w1r_ref[...], preferred_element_type=F32) + w1c_ref[...]
    else:
        trow_ref, tcol_ref = rest
        h1_ref[...] = h1
        pr = jnp.dot(h1, w1r_ref[...], preferred_element_type=F32)
        pc = jnp.dot(h1, w1c_ref[...], preferred_element_type=F32)
        trow_ref[...] = jnp.concatenate([pr, cp1], axis=1)
        tcol_ref[...] = jnp.concatenate([pc, -cp1], axis=1)


def _wspec(r, c):
    return pl.BlockSpec((r, c), lambda i: (0, 0))


def _tc_init(h_pad, cp, wemb, bemb, w1r, w1c):
    grid = (N_PAD // BN,)
    return pl.pallas_call(
        _init_body,
        grid=grid,
        in_specs=[
            pl.BlockSpec((BN, H), lambda i: (i, 0)),
            pl.BlockSpec((BN, 16), lambda i: (i, 0)),
            _wspec(H, H), _wspec(1, H), _wspec(H, H), _wspec(H, H),
        ],
        out_specs=[
            pl.BlockSpec((BN, H), lambda i: (i, 0)),
            pl.BlockSpec((BN, REC), lambda i: (i, 0)),
            pl.BlockSpec((BN, REC), lambda i: (i, 0)),
        ],
        out_shape=[
            jax.ShapeDtypeStruct((N_PAD, H), F32),
            jax.ShapeDtypeStruct((N_PAD, REC), F32),
            jax.ShapeDtypeStruct((N_PAD, REC), F32),
        ],
    )(h_pad, cp, wemb, bemb, w1r, w1c)


def _tc_edge(s, ea, w1ea, b1, wr, w2, b2, wc1, bc1, wc2):
    grid = (E_PAD // BE,)
    return pl.pallas_call(
        _edge_body,
        grid=grid,
        in_specs=[
            pl.BlockSpec((BE, REC), lambda i: (i, 0)),
            pl.BlockSpec((BE, DE), lambda i: (i, 0)),
            _wspec(DE, H), _wspec(1, H), _wspec(1, H),
            _wspec(H, H), _wspec(1, H),
            _wspec(H, H), _wspec(1, H), _wspec(1, H),
        ],
        out_specs=pl.BlockSpec((BE, REC), lambda i: (i, 0)),
        out_shape=jax.ShapeDtypeStruct((E_PAD, REC), F32),
    )(s, ea, w1ea, b1, wr, w2, b2, wc1, bc1, wc2)


def _tc_node(h, cp, p0, p1, wn1h, wn1m, bn1, wn2, bn2, w1r, w1c, last):
    grid = (N_PAD // BN,)
    out_specs = [
        pl.BlockSpec((BN, H), lambda i: (i, 0)),
        pl.BlockSpec((BN, 16), lambda i: (i, 0)),
    ]
    out_shape = [
        jax.ShapeDtypeStruct((N_PAD, H), F32),
        jax.ShapeDtypeStruct((N_PAD, 16), F32),
    ]
    if not last:
        out_specs += [pl.BlockSpec((BN, REC), lambda i: (i, 0))] * 2
        out_shape += [jax.ShapeDtypeStruct((N_PAD, REC), F32)] * 2
    return pl.pallas_call(
        functools.partial(_node_body, last=last),
        grid=grid,
        in_specs=[
            pl.BlockSpec((BN, H), lambda i: (i, 0)),
            pl.BlockSpec((BN, 16), lambda i: (i, 0)),
            pl.BlockSpec((BN, REC), lambda i: (i, 0)),
            pl.BlockSpec((BN, REC), lambda i: (i, 0)),
            _wspec(H, H), _wspec(H, H), _wspec(1, H),
            _wspec(H, H), _wspec(1, H), _wspec(H, H),
            _wspec(H, H) if not last else _wspec(1, H),
        ],
        out_specs=out_specs,
        out_shape=out_shape,
    )(h, cp, p0, p1, wn1h, wn1m, bn1, wn2, bn2, w1r, w1c)


def _sc_gather_body(trow, tcol, rowg, colg, out, idx_r, idx_c,
                    buf0, buf1, sr0, sr1, sc0, sc1, so0, so1):
    cid = lax.axis_index("c")
    sid = lax.axis_index("s")
    wid = sid * NC + cid
    base = wid * PER_W
    bufs, srs, scs, sos = (buf0, buf1), (sr0, sr1), (sc0, sc1), (so0, so1)

    pltpu.sync_copy(rowg.at[pl.ds(base, PER_W)], idx_r)
    pltpu.sync_copy(colg.at[pl.ds(base, PER_W)], idx_c)

    def row_start(j, b):
        pltpu.async_copy(trow.at[idx_r.at[pl.ds(j * CH, CH)]], bufs[b], srs[b])

    row_start(0, 0)

    def pair(jj, carry):
        for b in (0, 1):
            j = 2 * jj + b
            nb = 1 - b

            @pl.when(j >= 1)
            def _():
                pltpu.make_async_copy(bufs[nb], out.at[pl.ds(base, CH)], sos[nb]).wait()

            @pl.when(j + 1 < ITERS)
            def _():
                row_start(j + 1, nb)

            pltpu.make_async_copy(trow.at[idx_r.at[pl.ds(0, CH)]], bufs[b], srs[b]).wait()
            pltpu.async_copy(tcol.at[idx_c.at[pl.ds(j * CH, CH)]], bufs[b], scs[b],
                             add=True)
            pltpu.make_async_copy(tcol.at[idx_c.at[pl.ds(0, CH)]], bufs[b], scs[b]).wait()
            pltpu.async_copy(bufs[b], out.at[pl.ds(base + j * CH, CH)], sos[b])
        return carry

    lax.fori_loop(0, ITERS // 2, pair, 0)
    pltpu.make_async_copy(bufs[1], out.at[pl.ds(base, CH)], sos[1]).wait()


def _sc_gather(trow, tcol, rowg, colg):
    mesh = plsc.VectorSubcoreMesh(core_axis_name="c", subcore_axis_name="s")
    f = functools.partial(
        pl.kernel,
        out_type=jax.ShapeDtypeStruct((E_PAD, REC), F32),
        mesh=mesh,
        compiler_params=pltpu.CompilerParams(use_tc_tiling_on_sc=False),
        scratch_types=[
            pltpu.VMEM((PER_W,), I32),
            pltpu.VMEM((PER_W,), I32),
            pltpu.VMEM((CH, REC), F32),
            pltpu.VMEM((CH, REC), F32),
            pltpu.SemaphoreType.DMA,
            pltpu.SemaphoreType.DMA,
            pltpu.SemaphoreType.DMA,
            pltpu.SemaphoreType.DMA,
            pltpu.SemaphoreType.DMA,
            pltpu.SemaphoreType.DMA,
        ],
    )(_sc_gather_body)
    return f(trow, tcol, rowg, colg)


def _sc_scatter_body(edat, rows2, zeros, out, idx2, buf0, buf1, acc,
                     si0, si1, ss0, ss1):
    cid = lax.axis_index("c")
    sid = lax.axis_index("s")
    wid = sid * NC + cid
    base = wid * PER_W
    bufs, sis, sss = (buf0, buf1), (si0, si1), (ss0, ss1)

    pltpu.sync_copy(zeros.at[pl.ds(sid * RPT, RPT)], acc.at[pl.ds(sid * RPT, RPT)])
    pltpu.sync_copy(rows2.at[pl.ds(wid * ITERS_S, ITERS_S)], idx2)
    plsc.subcore_barrier()

    def in_start(j, b):
        pltpu.async_copy(edat.at[pl.ds(base + j * CH_S, CH_S)], bufs[b], sis[b])

    in_start(0, 0)

    def pair(jj, carry):
        for b in (0, 1):
            j = 2 * jj + b
            nb = 1 - b

            @pl.when(j >= 1)
            def _():
                pltpu.make_async_copy(bufs[nb], acc.at[idx2.at[0]], sss[nb]).wait()

            @pl.when(j + 1 < ITERS_S)
            def _():
                in_start(j + 1, nb)

            pltpu.make_async_copy(edat.at[pl.ds(base, CH_S)], bufs[b], sis[b]).wait()
            pltpu.async_copy(bufs[b], acc.at[idx2.at[j]], sss[b], add=True)
        return carry

    lax.fori_loop(0, ITERS_S // 2, pair, 0)
    pltpu.make_async_copy(bufs[1], acc.at[idx2.at[0]], sss[1]).wait()
    plsc.subcore_barrier()
    pltpu.sync_copy(acc.at[pl.ds(sid * RPT, RPT)], out.at[cid].at[pl.ds(sid * RPT, RPT)])


def _sc_scatter(edat, rows2, zeros):
    mesh = plsc.VectorSubcoreMesh(core_axis_name="c", subcore_axis_name="s")
    f = functools.partial(
        pl.kernel,
        out_type=jax.ShapeDtypeStruct((NC, N_PAD, REC), F32),
        mesh=mesh,
        compiler_params=pltpu.CompilerParams(use_tc_tiling_on_sc=False),
        scratch_types=[
            pltpu.VMEM((ITERS_S, CH_S), I32),
            pltpu.VMEM((CH_S, REC), F32),
            pltpu.VMEM((CH_S, REC), F32),
            pltpu.VMEM_SHARED((N_PAD, REC), F32),
            pltpu.SemaphoreType.DMA,
            pltpu.SemaphoreType.DMA,
            pltpu.SemaphoreType.DMA,
            pltpu.SemaphoreType.DMA,
        ],
    )(_sc_scatter_body)
    return f(edat, rows2, zeros)


def kernel(h, x, edges, edge_attr, params):
    row = edges[0]
    col = edges[1]

    h_pad = jnp.pad(h, ((0, N_PAD - N), (0, 0)))
    cp = jnp.pad(x, ((0, N_PAD - N), (0, 16 - 3)))
    ea = jnp.pad(edge_attr, ((0, E_PAD - E), (0, 0)))
    rowg = jnp.pad(row, (0, E_PAD - E))
    colg = jnp.pad(col, (0, E_PAD - E))
    rows_sc = jnp.pad(row, (0, E_PAD - E), constant_values=N)
    rows_sc2 = rows_sc.reshape(NW * ITERS_S, CH_S)
    zeros = jnp.zeros((N_PAD, REC), F32)

    def lw(i):
        p = params["layers"][i]
        w1 = p["e1"]["W"]
        return {
            "w1r": w1[:, :H].T, "w1c": w1[:, H:2 * H].T,
            "wr": w1[:, 2 * H].reshape(1, H), "w1ea": w1[:, 2 * H + 1:].T,
            "b1": p["e1"]["b"].reshape(1, H),
            "w2": p["e2"]["W"].T, "b2": p["e2"]["b"].reshape(1, H),
            "wc1": p["c1"]["W"].T, "bc1": p["c1"]["b"].reshape(1, H),
            "wc2": p["c2"]["W"].reshape(1, H),
            "wn1h": p["n1"]["W"][:, :H].T, "wn1m": p["n1"]["W"][:, H:].T,
            "bn1": p["n1"]["b"].reshape(1, H),
            "wn2": p["n2"]["W"].T, "bn2": p["n2"]["b"].reshape(1, H),
        }

    lws = [lw(i) for i in range(3)]
    wemb_in = params["emb_in"]["W"].T
    bemb_in = params["emb_in"]["b"].reshape(1, H)
    wemb_out = params["emb_out"]["W"].T
    bemb_out = params["emb_out"]["b"].reshape(1, H)

    hcur, trow, tcol = _tc_init(h_pad, cp, wemb_in, bemb_in,
                                lws[0]["w1r"], lws[0]["w1c"])
    for i in range(3):
        w = lws[i]
        s = _sc_gather(trow, tcol, rowg, colg)
        edat = _tc_edge(s, ea, w["w1ea"], w["b1"], w["wr"], w["w2"], w["b2"],
                        w["wc1"], w["bc1"], w["wc2"])
        parts = _sc_scatter(edat, rows_sc2, zeros)
        last = i == 2
        if last:
            nw1r, nw1c = wemb_out, bemb_out
        else:
            nw1r, nw1c = lws[i + 1]["w1r"], lws[i + 1]["w1c"]
        res = _tc_node(hcur, cp, parts[0], parts[1],
                       w["wn1h"], w["wn1m"], w["bn1"], w["wn2"], w["bn2"],
                       nw1r, nw1c, last)
        if last:
            hcur, cp = res
        else:
            hcur, cp, trow, tcol = res

    return hcur[:N], cp[:N, :3]


# speedup vs baseline: 2.3828x; 1.0674x over previous
import functools

import jax
import jax.numpy as jnp
from jax import lax
from jax.experimental import pallas as pl
from jax.experimental.pallas import tpu as pltpu
from jax.experimental.pallas import tpu_sc as plsc

F32 = jnp.float32
I32 = jnp.int32

N = 10000
E = 320000
H = 128
DE = 16
REC = H + 16
EPSV = 1e-8

NC, NS = 2, 16
NW = NC * NS
N_PAD = 10016
E_PAD = 327680
PER_W = E_PAD // NW
CH = 128
ITERS = PER_W // CH
CH_S = 64
ITERS_S = PER_W // CH_S
RPT = N_PAD // NS

BE = 2048
BN = 2504


def _silu(v):
    return v * jax.nn.sigmoid(v)


def _init_body(h_ref, cp_ref, wemb_ref, bemb_ref, w1r_ref, w1c_ref,
               h0_ref, trow_ref, tcol_ref):
    h0 = jnp.dot(h_ref[...], wemb_ref[...], preferred_element_type=F32) + bemb_ref[...]
    h0_ref[...] = h0
    cp = cp_ref[...]
    pr = jnp.dot(h0, w1r_ref[...], preferred_element_type=F32)
    pc = jnp.dot(h0, w1c_ref[...], preferred_element_type=F32)
    trow_ref[...] = jnp.concatenate([pr, cp], axis=1)
    tcol_ref[...] = jnp.concatenate([pc, -cp], axis=1)


def _edge_body(s_ref, ea_ref, w1ea_ref, b1_ref, wr_ref, w2_ref, b2_ref,
               wc1_ref, bc1_ref, wc2_ref, out_ref):
    s = s_ref[...]
    sh = s[:, :H]
    diff = s[:, H:]
    radial = jnp.sum(diff * diff, axis=1, keepdims=True)
    norm = jnp.sqrt(radial) + EPSV
    unit = diff / norm
    e1 = sh + radial * wr_ref[...] + b1_ref[...]
    e1 = e1 + jnp.dot(ea_ref[...], w1ea_ref[...], preferred_element_type=F32)
    m = _silu(e1)
    m = _silu(jnp.dot(m, w2_ref[...], preferred_element_type=F32) + b2_ref[...])
    ch = _silu(jnp.dot(m, wc1_ref[...], preferred_element_type=F32) + bc1_ref[...])
    c = jnp.sum(ch * wc2_ref[...], axis=1, keepdims=True)
    trans = unit * c
    lane = lax.broadcasted_iota(I32, trans.shape, 1)
    trans = jnp.where(lane == 3, 1.0, trans)
    out_ref[...] = jnp.concatenate([m, trans], axis=1)


def _node_body(h_ref, cp_ref, p0_ref, p1_ref, wn1h_ref, wn1m_ref, bn1_ref,
               wn2_ref, bn2_ref, w1r_ref, w1c_ref, h1_ref, cp1_ref,
               *rest, last):
    p0 = p0_ref[...]
    p1 = p1_ref[...]
    magg = p0[:, :H] + p1[:, :H]
    tail = p0[:, H:] + p1[:, H:]
    cnt = jnp.maximum(tail[:, 3:4], 1.0)
    lane = lax.broadcasted_iota(I32, tail.shape, 1)
    aggt = jnp.where(lane < 3, tail, 0.0)
    cp1 = cp_ref[...] + aggt / cnt
    cp1_ref[...] = cp1
    h = h_ref[...]
    o = _silu(jnp.dot(h, wn1h_ref[...], preferred_element_type=F32)
              + jnp.dot(magg, wn1m_ref[...], preferred_element_type=F32)
              + bn1_ref[...])
    o = jnp.dot(o, wn2_ref[...], preferred_element_type=F32) + bn2_ref[...]
    h1 = h + o
    if last:
        h1_ref[...] = jnp.dot(h1, w1r_ref[...], preferred_element_type=F32) + w1c_ref[...]
    else:
        trow_ref, tcol_ref = rest
        h1_ref[...] = h1
        pr = jnp.dot(h1, w1r_ref[...], preferred_element_type=F32)
        pc = jnp.dot(h1, w1c_ref[...], preferred_element_type=F32)
        trow_ref[...] = jnp.concatenate([pr, cp1], axis=1)
        tcol_ref[...] = jnp.concatenate([pc, -cp1], axis=1)


def _wspec(r, c):
    return pl.BlockSpec((r, c), lambda i: (0, 0))


def _tc_init(h_pad, cp, wemb, bemb, w1r, w1c):
    grid = (N_PAD // BN,)
    return pl.pallas_call(
        _init_body,
        grid=grid,
        in_specs=[
            pl.BlockSpec((BN, H), lambda i: (i, 0)),
            pl.BlockSpec((BN, 16), lambda i: (i, 0)),
            _wspec(H, H), _wspec(1, H), _wspec(H, H), _wspec(H, H),
        ],
        out_specs=[
            pl.BlockSpec((BN, H), lambda i: (i, 0)),
            pl.BlockSpec((BN, REC), lambda i: (i, 0)),
            pl.BlockSpec((BN, REC), lambda i: (i, 0)),
        ],
        out_shape=[
            jax.ShapeDtypeStruct((N_PAD, H), F32),
            jax.ShapeDtypeStruct((N_PAD, REC), F32),
            jax.ShapeDtypeStruct((N_PAD, REC), F32),
        ],
    )(h_pad, cp, wemb, bemb, w1r, w1c)


def _tc_edge(s, ea, w1ea, b1, wr, w2, b2, wc1, bc1, wc2):
    grid = (E_PAD // BE,)
    return pl.pallas_call(
        _edge_body,
        grid=grid,
        in_specs=[
            pl.BlockSpec((BE, REC), lambda i: (i, 0)),
            pl.BlockSpec((BE, DE), lambda i: (i, 0)),
            _wspec(DE, H), _wspec(1, H), _wspec(1, H),
            _wspec(H, H), _wspec(1, H),
            _wspec(H, H), _wspec(1, H), _wspec(1, H),
        ],
        out_specs=pl.BlockSpec((BE, REC), lambda i: (i, 0)),
        out_shape=jax.ShapeDtypeStruct((E_PAD, REC), F32),
    )(s, ea, w1ea, b1, wr, w2, b2, wc1, bc1, wc2)


def _tc_node(h, cp, p0, p1, wn1h, wn1m, bn1, wn2, bn2, w1r, w1c, last):
    grid = (N_PAD // BN,)
    out_specs = [
        pl.BlockSpec((BN, H), lambda i: (i, 0)),
        pl.BlockSpec((BN, 16), lambda i: (i, 0)),
    ]
    out_shape = [
        jax.ShapeDtypeStruct((N_PAD, H), F32),
        jax.ShapeDtypeStruct((N_PAD, 16), F32),
    ]
    if not last:
        out_specs += [pl.BlockSpec((BN, REC), lambda i: (i, 0))] * 2
        out_shape += [jax.ShapeDtypeStruct((N_PAD, REC), F32)] * 2
    return pl.pallas_call(
        functools.partial(_node_body, last=last),
        grid=grid,
        in_specs=[
            pl.BlockSpec((BN, H), lambda i: (i, 0)),
            pl.BlockSpec((BN, 16), lambda i: (i, 0)),
            pl.BlockSpec((BN, REC), lambda i: (i, 0)),
            pl.BlockSpec((BN, REC), lambda i: (i, 0)),
            _wspec(H, H), _wspec(H, H), _wspec(1, H),
            _wspec(H, H), _wspec(1, H), _wspec(H, H),
            _wspec(H, H) if not last else _wspec(1, H),
        ],
        out_specs=out_specs,
        out_shape=out_shape,
    )(h, cp, p0, p1, wn1h, wn1m, bn1, wn2, bn2, w1r, w1c)


NBUF = 4


def _sc_gather_body(trow, tcol, rowg, colg, out, idx_r, idx_c, *refs):
    bufs = refs[:NBUF]
    srs = refs[NBUF:2 * NBUF]
    scs = refs[2 * NBUF:3 * NBUF]
    sos = refs[3 * NBUF:4 * NBUF]
    cid = lax.axis_index("c")
    sid = lax.axis_index("s")
    wid = sid * NC + cid
    base = wid * PER_W

    pltpu.sync_copy(rowg.at[pl.ds(base, PER_W)], idx_r)
    pltpu.sync_copy(colg.at[pl.ds(base, PER_W)], idx_c)

    def slot(jj, carry):
        for b in range(NBUF):
            j = NBUF * jj + b

            @pl.when(jnp.logical_and(j >= NBUF, j < ITERS))
            def _():
                pltpu.make_async_copy(bufs[b], out.at[pl.ds(base, CH)], sos[b]).wait()

            @pl.when(j < ITERS)
            def _():
                pltpu.async_copy(trow.at[idx_r.at[pl.ds(j * CH, CH)]], bufs[b], srs[b])

            jB = j - 1
            bB = (b - 1) % NBUF

            @pl.when(jnp.logical_and(jB >= 0, jB < ITERS))
            def _():
                pltpu.make_async_copy(trow.at[idx_r.at[pl.ds(0, CH)]], bufs[bB],
                                      srs[bB]).wait()
                pltpu.async_copy(tcol.at[idx_c.at[pl.ds(jB * CH, CH)]], bufs[bB],
                                 scs[bB], add=True)

            jC = j - 2
            bC = (b - 2) % NBUF

            @pl.when(jnp.logical_and(jC >= 0, jC < ITERS))
            def _():
                pltpu.make_async_copy(tcol.at[idx_c.at[pl.ds(0, CH)]], bufs[bC],
                                      scs[bC]).wait()
                pltpu.async_copy(bufs[bC], out.at[pl.ds(base + jC * CH, CH)], sos[bC])
        return carry

    n_slots = ITERS + 2
    lax.fori_loop(0, (n_slots + NBUF - 1) // NBUF, slot, 0)
    for k in range(NBUF):
        b = (ITERS - NBUF + k) % NBUF
        pltpu.make_async_copy(bufs[b], out.at[pl.ds(base, CH)], sos[b]).wait()


def _sc_gather(trow, tcol, rowg, colg):
    mesh = plsc.VectorSubcoreMesh(core_axis_name="c", subcore_axis_name="s")
    f = functools.partial(
        pl.kernel,
        out_type=jax.ShapeDtypeStruct((E_PAD, REC), F32),
        mesh=mesh,
        compiler_params=pltpu.CompilerParams(use_tc_tiling_on_sc=False),
        scratch_types=[
            pltpu.VMEM((PER_W,), I32),
            pltpu.VMEM((PER_W,), I32),
        ] + [pltpu.VMEM((CH, REC), F32)] * NBUF
          + [pltpu.SemaphoreType.DMA] * (3 * NBUF),
    )(_sc_gather_body)
    return f(trow, tcol, rowg, colg)


NBUF_S = 3


def _sc_scatter_body(edat, rows2, zeros, out, idx2, *refs):
    bufs = refs[:NBUF_S]
    sis = refs[NBUF_S + 1:2 * NBUF_S + 1]
    sss = refs[2 * NBUF_S + 1:3 * NBUF_S + 1]
    acc = refs[NBUF_S]
    cid = lax.axis_index("c")
    sid = lax.axis_index("s")
    wid = sid * NC + cid
    base = wid * PER_W

    pltpu.sync_copy(zeros.at[pl.ds(sid * RPT, RPT)], acc.at[pl.ds(sid * RPT, RPT)])
    pltpu.sync_copy(rows2.at[pl.ds(wid * ITERS_S, ITERS_S)], idx2)
    plsc.subcore_barrier()

    def slot(jj, carry):
        for b in range(NBUF_S):
            j = NBUF_S * jj + b

            @pl.when(jnp.logical_and(j >= NBUF_S, j < ITERS_S))
            def _():
                pltpu.make_async_copy(bufs[b], acc.at[idx2.at[0]], sss[b]).wait()

            @pl.when(j < ITERS_S)
            def _():
                pltpu.async_copy(edat.at[pl.ds(base + j * CH_S, CH_S)], bufs[b],
                                 sis[b])

            jB = j - 1
            bB = (b - 1) % NBUF_S

            @pl.when(jnp.logical_and(jB >= 0, jB < ITERS_S))
            def _():
                pltpu.make_async_copy(edat.at[pl.ds(base, CH_S)], bufs[bB],
                                      sis[bB]).wait()
                pltpu.async_copy(bufs[bB], acc.at[idx2.at[jB]], sss[bB], add=True)
        return carry

    n_slots = ITERS_S + 1
    lax.fori_loop(0, (n_slots + NBUF_S - 1) // NBUF_S, slot, 0)
    for k in range(NBUF_S):
        b = (ITERS_S - NBUF_S + k) % NBUF_S
        pltpu.make_async_copy(bufs[b], acc.at[idx2.at[0]], sss[b]).wait()
    plsc.subcore_barrier()
    pltpu.sync_copy(acc.at[pl.ds(sid * RPT, RPT)], out.at[cid].at[pl.ds(sid * RPT, RPT)])


def _sc_scatter(edat, rows2, zeros):
    mesh = plsc.VectorSubcoreMesh(core_axis_name="c", subcore_axis_name="s")
    f = functools.partial(
        pl.kernel,
        out_type=jax.ShapeDtypeStruct((NC, N_PAD, REC), F32),
        mesh=mesh,
        compiler_params=pltpu.CompilerParams(use_tc_tiling_on_sc=False),
        scratch_types=[
            pltpu.VMEM((ITERS_S, CH_S), I32),
        ] + [pltpu.VMEM((CH_S, REC), F32)] * NBUF_S
          + [pltpu.VMEM_SHARED((N_PAD, REC), F32)]
          + [pltpu.SemaphoreType.DMA] * (2 * NBUF_S),
    )(_sc_scatter_body)
    return f(edat, rows2, zeros)


def kernel(h, x, edges, edge_attr, params):
    row = edges[0]
    col = edges[1]

    h_pad = jnp.pad(h, ((0, N_PAD - N), (0, 0)))
    cp = jnp.pad(x, ((0, N_PAD - N), (0, 16 - 3)))
    ea = jnp.pad(edge_attr, ((0, E_PAD - E), (0, 0)))
    rowg = jnp.pad(row, (0, E_PAD - E))
    colg = jnp.pad(col, (0, E_PAD - E))
    rows_sc = jnp.pad(row, (0, E_PAD - E), constant_values=N)
    rows_sc2 = rows_sc.reshape(NW * ITERS_S, CH_S)
    zeros = jnp.zeros((N_PAD, REC), F32)

    def lw(i):
        p = params["layers"][i]
        w1 = p["e1"]["W"]
        return {
            "w1r": w1[:, :H].T, "w1c": w1[:, H:2 * H].T,
            "wr": w1[:, 2 * H].reshape(1, H), "w1ea": w1[:, 2 * H + 1:].T,
            "b1": p["e1"]["b"].reshape(1, H),
            "w2": p["e2"]["W"].T, "b2": p["e2"]["b"].reshape(1, H),
            "wc1": p["c1"]["W"].T, "bc1": p["c1"]["b"].reshape(1, H),
            "wc2": p["c2"]["W"].reshape(1, H),
            "wn1h": p["n1"]["W"][:, :H].T, "wn1m": p["n1"]["W"][:, H:].T,
            "bn1": p["n1"]["b"].reshape(1, H),
            "wn2": p["n2"]["W"].T, "bn2": p["n2"]["b"].reshape(1, H),
        }

    lws = [lw(i) for i in range(3)]
    wemb_in = params["emb_in"]["W"].T
    bemb_in = params["emb_in"]["b"].reshape(1, H)
    wemb_out = params["emb_out"]["W"].T
    bemb_out = params["emb_out"]["b"].reshape(1, H)

    hcur, trow, tcol = _tc_init(h_pad, cp, wemb_in, bemb_in,
                                lws[0]["w1r"], lws[0]["w1c"])
    for i in range(3):
        w = lws[i]
        s = _sc_gather(trow, tcol, rowg, colg)
        edat = _tc_edge(s, ea, w["w1ea"], w["b1"], w["wr"], w["w2"], w["b2"],
                        w["wc1"], w["bc1"], w["wc2"])
        parts = _sc_scatter(edat, rows_sc2, zeros)
        last = i == 2
        if last:
            nw1r, nw1c = wemb_out, bemb_out
        else:
            nw1r, nw1c = lws[i + 1]["w1r"], lws[i + 1]["w1c"]
        res = _tc_node(hcur, cp, parts[0], parts[1],
                       w["wn1h"], w["wn1m"], w["bn1"], w["wn2"], w["bn2"],
                       nw1r, nw1c, last)
        if last:
            hcur, cp = res
        else:
            hcur, cp, trow, tcol = res

    return hcur[:N], cp[:N, :3]


# speedup vs baseline: 2.5101x; 1.0534x over previous
import functools

import jax
import jax.numpy as jnp
from jax import lax
from jax.experimental import pallas as pl
from jax.experimental.pallas import tpu as pltpu
from jax.experimental.pallas import tpu_sc as plsc

F32 = jnp.float32
I32 = jnp.int32

N = 10000
E = 320000
H = 128
DE = 16
REC = H + 16
EPSV = 1e-8

NC, NS = 2, 16
NW = NC * NS
N_PAD = 10016
E_PAD = 327680
PER_W = E_PAD // NW
CH = 128
ITERS = PER_W // CH
CH_S = 64
ITERS_S = PER_W // CH_S
RPT = N_PAD // NS

BE = 2048
BN = 2504


def _silu(v):
    return v * jax.nn.sigmoid(v)


def _init_body(h_ref, cp_ref, wemb_ref, bemb_ref, w1r_ref, w1c_ref,
               h0_ref, t_ref):
    h0 = jnp.dot(h_ref[...], wemb_ref[...], preferred_element_type=F32) + bemb_ref[...]
    h0_ref[...] = h0
    cp = cp_ref[...]
    pr = jnp.dot(h0, w1r_ref[...], preferred_element_type=F32)
    pc = jnp.dot(h0, w1c_ref[...], preferred_element_type=F32)
    t_ref[0] = jnp.concatenate([pr, cp], axis=1)
    t_ref[1] = jnp.concatenate([pc, -cp], axis=1)


def _edge_body(s0_ref, s1_ref, ea_ref, w1ea_ref, b1_ref, wr_ref, w2_ref, b2_ref,
               wc1_ref, bc1_ref, wc2_ref, out_ref):
    s = s0_ref[0] + s1_ref[0]
    sh = s[:, :H]
    diff = s[:, H:]
    radial = jnp.sum(diff * diff, axis=1, keepdims=True)
    norm = jnp.sqrt(radial) + EPSV
    unit = diff / norm
    e1 = sh + radial * wr_ref[...] + b1_ref[...]
    e1 = e1 + jnp.dot(ea_ref[...], w1ea_ref[...], preferred_element_type=F32)
    m = _silu(e1)
    m = _silu(jnp.dot(m, w2_ref[...], preferred_element_type=F32) + b2_ref[...])
    ch = _silu(jnp.dot(m, wc1_ref[...], preferred_element_type=F32) + bc1_ref[...])
    c = jnp.sum(ch * wc2_ref[...], axis=1, keepdims=True)
    trans = unit * c
    lane = lax.broadcasted_iota(I32, trans.shape, 1)
    trans = jnp.where(lane == 3, 1.0, trans)
    out_ref[...] = jnp.concatenate([m, trans], axis=1)


def _node_body(h_ref, cp_ref, p0_ref, p1_ref, wn1h_ref, wn1m_ref, bn1_ref,
               wn2_ref, bn2_ref, w1r_ref, w1c_ref, h1_ref, cp1_ref,
               *rest, last):
    p0 = p0_ref[...]
    p1 = p1_ref[...]
    magg = p0[:, :H] + p1[:, :H]
    tail = p0[:, H:] + p1[:, H:]
    cnt = jnp.maximum(tail[:, 3:4], 1.0)
    lane = lax.broadcasted_iota(I32, tail.shape, 1)
    aggt = jnp.where(lane < 3, tail, 0.0)
    cp1 = cp_ref[...] + aggt / cnt
    cp1_ref[...] = cp1
    h = h_ref[...]
    o = _silu(jnp.dot(h, wn1h_ref[...], preferred_element_type=F32)
              + jnp.dot(magg, wn1m_ref[...], preferred_element_type=F32)
              + bn1_ref[...])
    o = jnp.dot(o, wn2_ref[...], preferred_element_type=F32) + bn2_ref[...]
    h1 = h + o
    if last:
        h1_ref[...] = jnp.dot(h1, w1r_ref[...], preferred_element_type=F32) + w1c_ref[...]
    else:
        (t_ref,) = rest
        h1_ref[...] = h1
        pr = jnp.dot(h1, w1r_ref[...], preferred_element_type=F32)
        pc = jnp.dot(h1, w1c_ref[...], preferred_element_type=F32)
        t_ref[0] = jnp.concatenate([pr, cp1], axis=1)
        t_ref[1] = jnp.concatenate([pc, -cp1], axis=1)


def _wspec(r, c):
    return pl.BlockSpec((r, c), lambda i: (0, 0))


def _tc_init(h_pad, cp, wemb, bemb, w1r, w1c):
    grid = (N_PAD // BN,)
    return pl.pallas_call(
        _init_body,
        grid=grid,
        in_specs=[
            pl.BlockSpec((BN, H), lambda i: (i, 0)),
            pl.BlockSpec((BN, 16), lambda i: (i, 0)),
            _wspec(H, H), _wspec(1, H), _wspec(H, H), _wspec(H, H),
        ],
        out_specs=[
            pl.BlockSpec((BN, H), lambda i: (i, 0)),
            pl.BlockSpec((2, BN, REC), lambda i: (0, i, 0)),
        ],
        out_shape=[
            jax.ShapeDtypeStruct((N_PAD, H), F32),
            jax.ShapeDtypeStruct((2, N_PAD, REC), F32),
        ],
    )(h_pad, cp, wemb, bemb, w1r, w1c)


def _tc_edge(s, ea, w1ea, b1, wr, w2, b2, wc1, bc1, wc2):
    grid = (E_PAD // BE,)
    return pl.pallas_call(
        _edge_body,
        grid=grid,
        in_specs=[
            pl.BlockSpec((1, BE, REC), lambda i: (0, i, 0)),
            pl.BlockSpec((1, BE, REC), lambda i: (1, i, 0)),
            pl.BlockSpec((BE, DE), lambda i: (i, 0)),
            _wspec(DE, H), _wspec(1, H), _wspec(1, H),
            _wspec(H, H), _wspec(1, H),
            _wspec(H, H), _wspec(1, H), _wspec(1, H),
        ],
        out_specs=pl.BlockSpec((BE, REC), lambda i: (i, 0)),
        out_shape=jax.ShapeDtypeStruct((E_PAD, REC), F32),
    )(s, s, ea, w1ea, b1, wr, w2, b2, wc1, bc1, wc2)


def _tc_node(h, cp, p0, p1, wn1h, wn1m, bn1, wn2, bn2, w1r, w1c, last):
    grid = (N_PAD // BN,)
    out_specs = [
        pl.BlockSpec((BN, H), lambda i: (i, 0)),
        pl.BlockSpec((BN, 16), lambda i: (i, 0)),
    ]
    out_shape = [
        jax.ShapeDtypeStruct((N_PAD, H), F32),
        jax.ShapeDtypeStruct((N_PAD, 16), F32),
    ]
    if not last:
        out_specs += [pl.BlockSpec((2, BN, REC), lambda i: (0, i, 0))]
        out_shape += [jax.ShapeDtypeStruct((2, N_PAD, REC), F32)]
    return pl.pallas_call(
        functools.partial(_node_body, last=last),
        grid=grid,
        in_specs=[
            pl.BlockSpec((BN, H), lambda i: (i, 0)),
            pl.BlockSpec((BN, 16), lambda i: (i, 0)),
            pl.BlockSpec((BN, REC), lambda i: (i, 0)),
            pl.BlockSpec((BN, REC), lambda i: (i, 0)),
            _wspec(H, H), _wspec(H, H), _wspec(1, H),
            _wspec(H, H), _wspec(1, H), _wspec(H, H),
            _wspec(H, H) if not last else _wspec(1, H),
        ],
        out_specs=out_specs,
        out_shape=out_shape,
    )(h, cp, p0, p1, wn1h, wn1m, bn1, wn2, bn2, w1r, w1c)


CH2 = 64
PER_T = E_PAD // NS
ITERS2 = PER_T // CH2


def _sc_gather_body(tables, eidx, out, idx_v, buf0, buf1, tbl, sg0, sg1, so0, so1):
    cid = lax.axis_index("c")
    sid = lax.axis_index("s")
    tbase = sid * PER_T
    bufs, sgs, sos = (buf0, buf1), (sg0, sg1), (so0, so1)

    pltpu.sync_copy(tables.at[cid].at[pl.ds(sid * RPT, RPT)],
                    tbl.at[pl.ds(sid * RPT, RPT)])
    pltpu.sync_copy(eidx.at[cid].at[pl.ds(tbase, PER_T)], idx_v)
    plsc.subcore_barrier()

    def slot(jj, carry):
        for b in (0, 1):
            j = 2 * jj + b

            @pl.when(jnp.logical_and(j >= 2, j < ITERS2))
            def _():
                pltpu.make_async_copy(bufs[b], out.at[0].at[pl.ds(0, CH2)],
                                      sos[b]).wait()

            @pl.when(j < ITERS2)
            def _():
                pltpu.async_copy(tbl.at[idx_v.at[pl.ds(j * CH2, CH2)]], bufs[b],
                                 sgs[b])

            jB = j - 1
            bB = 1 - b

            @pl.when(jnp.logical_and(jB >= 0, jB < ITERS2))
            def _():
                pltpu.make_async_copy(tbl.at[idx_v.at[pl.ds(0, CH2)]], bufs[bB],
                                      sgs[bB]).wait()
                pltpu.async_copy(bufs[bB], out.at[cid].at[pl.ds(tbase + jB * CH2, CH2)],
                                 sos[bB])
        return carry

    n_slots = ITERS2 + 1
    lax.fori_loop(0, (n_slots + 1) // 2, slot, 0)
    for b in (0, 1):
        pltpu.make_async_copy(bufs[b], out.at[0].at[pl.ds(0, CH2)], sos[b]).wait()


def _sc_gather(tables, eidx):
    mesh = plsc.VectorSubcoreMesh(core_axis_name="c", subcore_axis_name="s")
    f = functools.partial(
        pl.kernel,
        out_type=jax.ShapeDtypeStruct((NC, E_PAD, REC), F32),
        mesh=mesh,
        compiler_params=pltpu.CompilerParams(use_tc_tiling_on_sc=False),
        scratch_types=[
            pltpu.VMEM((PER_T,), I32),
            pltpu.VMEM((CH2, REC), F32),
            pltpu.VMEM((CH2, REC), F32),
            pltpu.VMEM_SHARED((N_PAD, REC), F32),
            pltpu.SemaphoreType.DMA,
            pltpu.SemaphoreType.DMA,
            pltpu.SemaphoreType.DMA,
            pltpu.SemaphoreType.DMA,
        ],
    )(_sc_gather_body)
    return f(tables, eidx)


NBUF_S = 3


def _sc_scatter_body(edat, rows2, zeros, out, idx2, *refs):
    bufs = refs[:NBUF_S]
    sis = refs[NBUF_S + 1:2 * NBUF_S + 1]
    sss = refs[2 * NBUF_S + 1:3 * NBUF_S + 1]
    acc = refs[NBUF_S]
    cid = lax.axis_index("c")
    sid = lax.axis_index("s")
    wid = sid * NC + cid
    base = wid * PER_W

    pltpu.sync_copy(zeros.at[pl.ds(sid * RPT, RPT)], acc.at[pl.ds(sid * RPT, RPT)])
    pltpu.sync_copy(rows2.at[pl.ds(wid * ITERS_S, ITERS_S)], idx2)
    plsc.subcore_barrier()

    def slot(jj, carry):
        for b in range(NBUF_S):
            j = NBUF_S * jj + b

            @pl.when(jnp.logical_and(j >= NBUF_S, j < ITERS_S))
            def _():
                pltpu.make_async_copy(bufs[b], acc.at[idx2.at[0]], sss[b]).wait()

            @pl.when(j < ITERS_S)
            def _():
                pltpu.async_copy(edat.at[pl.ds(base + j * CH_S, CH_S)], bufs[b],
                                 sis[b])

            jB = j - 1
            bB = (b - 1) % NBUF_S

            @pl.when(jnp.logical_and(jB >= 0, jB < ITERS_S))
            def _():
                pltpu.make_async_copy(edat.at[pl.ds(base, CH_S)], bufs[bB],
                                      sis[bB]).wait()
                pltpu.async_copy(bufs[bB], acc.at[idx2.at[jB]], sss[bB], add=True)
        return carry

    n_slots = ITERS_S + 1
    lax.fori_loop(0, (n_slots + NBUF_S - 1) // NBUF_S, slot, 0)
    for k in range(NBUF_S):
        b = (ITERS_S - NBUF_S + k) % NBUF_S
        pltpu.make_async_copy(bufs[b], acc.at[idx2.at[0]], sss[b]).wait()
    plsc.subcore_barrier()
    pltpu.sync_copy(acc.at[pl.ds(sid * RPT, RPT)], out.at[cid].at[pl.ds(sid * RPT, RPT)])


def _sc_scatter(edat, rows2, zeros):
    mesh = plsc.VectorSubcoreMesh(core_axis_name="c", subcore_axis_name="s")
    f = functools.partial(
        pl.kernel,
        out_type=jax.ShapeDtypeStruct((NC, N_PAD, REC), F32),
        mesh=mesh,
        compiler_params=pltpu.CompilerParams(use_tc_tiling_on_sc=False),
        scratch_types=[
            pltpu.VMEM((ITERS_S, CH_S), I32),
        ] + [pltpu.VMEM((CH_S, REC), F32)] * NBUF_S
          + [pltpu.VMEM_SHARED((N_PAD, REC), F32)]
          + [pltpu.SemaphoreType.DMA] * (2 * NBUF_S),
    )(_sc_scatter_body)
    return f(edat, rows2, zeros)


def kernel(h, x, edges, edge_attr, params):
    row = edges[0]
    col = edges[1]

    h_pad = jnp.pad(h, ((0, N_PAD - N), (0, 0)))
    cp = jnp.pad(x, ((0, N_PAD - N), (0, 16 - 3)))
    ea = jnp.pad(edge_attr, ((0, E_PAD - E), (0, 0)))
    rowg = jnp.pad(row, (0, E_PAD - E))
    colg = jnp.pad(col, (0, E_PAD - E))
    eidx = jnp.stack([rowg, colg])
    rows_sc = jnp.pad(row, (0, E_PAD - E), constant_values=N)
    rows_sc2 = rows_sc.reshape(NW * ITERS_S, CH_S)
    zeros = jnp.zeros((N_PAD, REC), F32)

    def lw(i):
        p = params["layers"][i]
        w1 = p["e1"]["W"]
        return {
            "w1r": w1[:, :H].T, "w1c": w1[:, H:2 * H].T,
            "wr": w1[:, 2 * H].reshape(1, H), "w1ea": w1[:, 2 * H + 1:].T,
            "b1": p["e1"]["b"].reshape(1, H),
            "w2": p["e2"]["W"].T, "b2": p["e2"]["b"].reshape(1, H),
            "wc1": p["c1"]["W"].T, "bc1": p["c1"]["b"].reshape(1, H),
            "wc2": p["c2"]["W"].reshape(1, H),
            "wn1h": p["n1"]["W"][:, :H].T, "wn1m": p["n1"]["W"][:, H:].T,
            "bn1": p["n1"]["b"].reshape(1, H),
            "wn2": p["n2"]["W"].T, "bn2": p["n2"]["b"].reshape(1, H),
        }

    lws = [lw(i) for i in range(3)]
    wemb_in = params["emb_in"]["W"].T
    bemb_in = params["emb_in"]["b"].reshape(1, H)
    wemb_out = params["emb_out"]["W"].T
    bemb_out = params["emb_out"]["b"].reshape(1, H)

    hcur, tables = _tc_init(h_pad, cp, wemb_in, bemb_in,
                            lws[0]["w1r"], lws[0]["w1c"])
    for i in range(3):
        w = lws[i]
        s = _sc_gather(tables, eidx)
        edat = _tc_edge(s, ea, w["w1ea"], w["b1"], w["wr"], w["w2"], w["b2"],
                        w["wc1"], w["bc1"], w["wc2"])
        parts = _sc_scatter(edat, rows_sc2, zeros)
        last = i == 2
        if last:
            nw1r, nw1c = wemb_out, bemb_out
        else:
            nw1r, nw1c = lws[i + 1]["w1r"], lws[i + 1]["w1c"]
        res = _tc_node(hcur, cp, parts[0], parts[1],
                       w["wn1h"], w["wn1m"], w["bn1"], w["wn2"], w["bn2"],
                       nw1r, nw1c, last)
        if last:
            hcur, cp = res
        else:
            hcur, cp, tables = res

    return hcur[:N], cp[:N, :3]


# speedup vs baseline: 4.1736x; 1.6627x over previous
import functools

import jax
import jax.numpy as jnp
from jax import lax
from jax.experimental import pallas as pl
from jax.experimental.pallas import tpu as pltpu
from jax.experimental.pallas import tpu_sc as plsc

F32 = jnp.float32
I32 = jnp.int32

N = 10000
E = 320000
H = 128
DE = 16
GW = 16
EPSV = 1e-8

NC, NS = 2, 16
NW = NC * NS
N_PAD = 10016
E_PAD = 327680
PER_T = E_PAD // NS
CH2 = 64
ITERS2 = PER_T // CH2
PER_W = E_PAD // NW
CH_S = 64
ITERS_S = PER_W // CH_S
RPT = N_PAD // NS

BE = 2048
BN = 2504


def _silu(v):
    return v * jax.nn.sigmoid(v)


def _init_body(h_ref, cp_ref, wemb_ref, bemb_ref, w1r_ref, w1c_ref,
               h0_ref, tp_ref, tg_ref):
    h0 = jnp.dot(h_ref[...], wemb_ref[...], preferred_element_type=F32) + bemb_ref[...]
    h0_ref[...] = h0
    cp = cp_ref[...]
    tp_ref[0] = jnp.dot(h0, w1r_ref[...], preferred_element_type=F32)
    tp_ref[1] = jnp.dot(h0, w1c_ref[...], preferred_element_type=F32)
    tg_ref[0] = cp
    tg_ref[1] = -cp


def _edge_body(sh0_ref, sh1_ref, sg0_ref, sg1_ref, ea_ref, w1ea_ref, b1_ref,
               wr_ref, w2_ref, b2_ref, wc1_ref, bc1_ref, wc2_ref,
               outh_ref, outg_ref):
    sh = sh0_ref[0] + sh1_ref[0]
    diff = sg0_ref[0] + sg1_ref[0]
    radial = jnp.sum(diff * diff, axis=1, keepdims=True)
    norm = jnp.sqrt(radial) + EPSV
    unit = diff / norm
    e1 = sh + radial * wr_ref[...] + b1_ref[...]
    e1 = e1 + jnp.dot(ea_ref[...], w1ea_ref[...], preferred_element_type=F32)
    m = _silu(e1)
    m = _silu(jnp.dot(m, w2_ref[...], preferred_element_type=F32) + b2_ref[...])
    ch = _silu(jnp.dot(m, wc1_ref[...], preferred_element_type=F32) + bc1_ref[...])
    c = jnp.sum(ch * wc2_ref[...], axis=1, keepdims=True)
    trans = unit * c
    lane = lax.broadcasted_iota(I32, trans.shape, 1)
    trans = jnp.where(lane == 3, 1.0, trans)
    outh_ref[...] = m
    outg_ref[...] = trans


def _node_body(h_ref, cp_ref, ph0_ref, ph1_ref, pg0_ref, pg1_ref,
               wn1h_ref, wn1m_ref, bn1_ref, wn2_ref, bn2_ref,
               w1r_ref, w1c_ref, h1_ref, cp1_ref, *rest, last):
    magg = ph0_ref[0] + ph1_ref[0]
    tail = pg0_ref[0] + pg1_ref[0]
    cnt = jnp.maximum(tail[:, 3:4], 1.0)
    lane = lax.broadcasted_iota(I32, tail.shape, 1)
    aggt = jnp.where(lane < 3, tail, 0.0)
    cp1 = cp_ref[...] + aggt / cnt
    cp1_ref[...] = cp1
    h = h_ref[...]
    o = _silu(jnp.dot(h, wn1h_ref[...], preferred_element_type=F32)
              + jnp.dot(magg, wn1m_ref[...], preferred_element_type=F32)
              + bn1_ref[...])
    o = jnp.dot(o, wn2_ref[...], preferred_element_type=F32) + bn2_ref[...]
    h1 = h + o
    if last:
        h1_ref[...] = jnp.dot(h1, w1r_ref[...], preferred_element_type=F32) + w1c_ref[...]
    else:
        tp_ref, tg_ref = rest
        h1_ref[...] = h1
        tp_ref[0] = jnp.dot(h1, w1r_ref[...], preferred_element_type=F32)
        tp_ref[1] = jnp.dot(h1, w1c_ref[...], preferred_element_type=F32)
        tg_ref[0] = cp1
        tg_ref[1] = -cp1


def _wspec(r, c):
    return pl.BlockSpec((r, c), lambda i: (0, 0))


def _tc_init(h_pad, cp, wemb, bemb, w1r, w1c):
    grid = (N_PAD // BN,)
    return pl.pallas_call(
        _init_body,
        grid=grid,
        in_specs=[
            pl.BlockSpec((BN, H), lambda i: (i, 0)),
            pl.BlockSpec((BN, GW), lambda i: (i, 0)),
            _wspec(H, H), _wspec(1, H), _wspec(H, H), _wspec(H, H),
        ],
        out_specs=[
            pl.BlockSpec((BN, H), lambda i: (i, 0)),
            pl.BlockSpec((2, BN, H), lambda i: (0, i, 0)),
            pl.BlockSpec((2, BN, GW), lambda i: (0, i, 0)),
        ],
        out_shape=[
            jax.ShapeDtypeStruct((N_PAD, H), F32),
            jax.ShapeDtypeStruct((2, N_PAD, H), F32),
            jax.ShapeDtypeStruct((2, N_PAD, GW), F32),
        ],
    )(h_pad, cp, wemb, bemb, w1r, w1c)


def _tc_edge(sh, sg, ea, w1ea, b1, wr, w2, b2, wc1, bc1, wc2):
    grid = (E_PAD // BE,)
    return pl.pallas_call(
        _edge_body,
        grid=grid,
        in_specs=[
            pl.BlockSpec((1, BE, H), lambda i: (0, i, 0)),
            pl.BlockSpec((1, BE, H), lambda i: (1, i, 0)),
            pl.BlockSpec((1, BE, GW), lambda i: (0, i, 0)),
            pl.BlockSpec((1, BE, GW), lambda i: (1, i, 0)),
            pl.BlockSpec((BE, DE), lambda i: (i, 0)),
            _wspec(DE, H), _wspec(1, H), _wspec(1, H),
            _wspec(H, H), _wspec(1, H),
            _wspec(H, H), _wspec(1, H), _wspec(1, H),
        ],
        out_specs=[
            pl.BlockSpec((BE, H), lambda i: (i, 0)),
            pl.BlockSpec((BE, GW), lambda i: (i, 0)),
        ],
        out_shape=[
            jax.ShapeDtypeStruct((E_PAD, H), F32),
            jax.ShapeDtypeStruct((E_PAD, GW), F32),
        ],
    )(sh, sh, sg, sg, ea, w1ea, b1, wr, w2, b2, wc1, bc1, wc2)


def _tc_node(h, cp, ph, pg, wn1h, wn1m, bn1, wn2, bn2, w1r, w1c, last):
    grid = (N_PAD // BN,)
    out_specs = [
        pl.BlockSpec((BN, H), lambda i: (i, 0)),
        pl.BlockSpec((BN, GW), lambda i: (i, 0)),
    ]
    out_shape = [
        jax.ShapeDtypeStruct((N_PAD, H), F32),
        jax.ShapeDtypeStruct((N_PAD, GW), F32),
    ]
    if not last:
        out_specs += [
            pl.BlockSpec((2, BN, H), lambda i: (0, i, 0)),
            pl.BlockSpec((2, BN, GW), lambda i: (0, i, 0)),
        ]
        out_shape += [
            jax.ShapeDtypeStruct((2, N_PAD, H), F32),
            jax.ShapeDtypeStruct((2, N_PAD, GW), F32),
        ]
    return pl.pallas_call(
        functools.partial(_node_body, last=last),
        grid=grid,
        in_specs=[
            pl.BlockSpec((BN, H), lambda i: (i, 0)),
            pl.BlockSpec((BN, GW), lambda i: (i, 0)),
            pl.BlockSpec((1, BN, H), lambda i: (0, i, 0)),
            pl.BlockSpec((1, BN, H), lambda i: (1, i, 0)),
            pl.BlockSpec((1, BN, GW), lambda i: (0, i, 0)),
            pl.BlockSpec((1, BN, GW), lambda i: (1, i, 0)),
            _wspec(H, H), _wspec(H, H), _wspec(1, H),
            _wspec(H, H), _wspec(1, H), _wspec(H, H),
            _wspec(H, H) if not last else _wspec(1, H),
        ],
        out_specs=out_specs,
        out_shape=out_shape,
    )(h, cp, ph, ph, pg, pg, wn1h, wn1m, bn1, wn2, bn2, w1r, w1c)


def _sc_gather_body(tp, tg, eidx, outh, outg, idx_v,
                    bp0, bp1, bg0, bg1, tblp, tblg,
                    sp0, sp1, sg0, sg1, sop0, sop1, sog0, sog1):
    cid = lax.axis_index("c")
    sid = lax.axis_index("s")
    tbase = sid * PER_T
    bps, bgs = (bp0, bp1), (bg0, bg1)
    sps, sgs = (sp0, sp1), (sg0, sg1)
    sops, sogs = (sop0, sop1), (sog0, sog1)

    pltpu.sync_copy(tp.at[cid].at[pl.ds(sid * RPT, RPT)],
                    tblp.at[pl.ds(sid * RPT, RPT)])
    pltpu.sync_copy(tg.at[cid].at[pl.ds(sid * RPT, RPT)],
                    tblg.at[pl.ds(sid * RPT, RPT)])
    pltpu.sync_copy(eidx.at[cid].at[pl.ds(tbase, PER_T)], idx_v)
    plsc.subcore_barrier()

    def slot(jj, carry):
        for b in (0, 1):
            j = 2 * jj + b

            @pl.when(jnp.logical_and(j >= 2, j < ITERS2))
            def _():
                pltpu.make_async_copy(bps[b], outh.at[0].at[pl.ds(0, CH2)],
                                      sops[b]).wait()
                pltpu.make_async_copy(bgs[b], outg.at[0].at[pl.ds(0, CH2)],
                                      sogs[b]).wait()

            @pl.when(j < ITERS2)
            def _():
                pltpu.async_copy(tblp.at[idx_v.at[pl.ds(j * CH2, CH2)]], bps[b],
                                 sps[b])
                pltpu.async_copy(tblg.at[idx_v.at[pl.ds(j * CH2, CH2)]], bgs[b],
                                 sgs[b])

            jB = j - 1
            bB = 1 - b

            @pl.when(jnp.logical_and(jB >= 0, jB < ITERS2))
            def _():
                pltpu.make_async_copy(tblp.at[idx_v.at[pl.ds(0, CH2)]], bps[bB],
                                      sps[bB]).wait()
                pltpu.make_async_copy(tblg.at[idx_v.at[pl.ds(0, CH2)]], bgs[bB],
                                      sgs[bB]).wait()
                pltpu.async_copy(bps[bB], outh.at[cid].at[pl.ds(tbase + jB * CH2, CH2)],
                                 sops[bB])
                pltpu.async_copy(bgs[bB], outg.at[cid].at[pl.ds(tbase + jB * CH2, CH2)],
                                 sogs[bB])
        return carry

    n_slots = ITERS2 + 1
    lax.fori_loop(0, (n_slots + 1) // 2, slot, 0)
    for b in (0, 1):
        pltpu.make_async_copy(bps[b], outh.at[0].at[pl.ds(0, CH2)], sops[b]).wait()
        pltpu.make_async_copy(bgs[b], outg.at[0].at[pl.ds(0, CH2)], sogs[b]).wait()


def _sc_gather(tp, tg, eidx):
    mesh = plsc.VectorSubcoreMesh(core_axis_name="c", subcore_axis_name="s")
    f = functools.partial(
        pl.kernel,
        out_type=[
            jax.ShapeDtypeStruct((NC, E_PAD, H), F32),
            jax.ShapeDtypeStruct((NC, E_PAD, GW), F32),
        ],
        mesh=mesh,
        compiler_params=pltpu.CompilerParams(use_tc_tiling_on_sc=False),
        scratch_types=[
            pltpu.VMEM((PER_T,), I32),
            pltpu.VMEM((CH2, H), F32),
            pltpu.VMEM((CH2, H), F32),
            pltpu.VMEM((CH2, GW), F32),
            pltpu.VMEM((CH2, GW), F32),
            pltpu.VMEM_SHARED((N_PAD, H), F32),
            pltpu.VMEM_SHARED((N_PAD, GW), F32),
        ] + [pltpu.SemaphoreType.DMA] * 8,
    )(_sc_gather_body)
    return f(tp, tg, eidx)


def _sc_scatter_body(edath, edatg, rows2, zerosh, zerosg, outh, outg, idx2,
                     bh0, bh1, bh2, bg0, bg1, bg2, acch, accg,
                     sih0, sih1, sih2, sig0, sig1, sig2,
                     ssh0, ssh1, ssh2, ssg0, ssg1, ssg2):
    cid = lax.axis_index("c")
    sid = lax.axis_index("s")
    wid = sid * NC + cid
    base = wid * PER_W
    bhs, bgs = (bh0, bh1, bh2), (bg0, bg1, bg2)
    sihs, sigs = (sih0, sih1, sih2), (sig0, sig1, sig2)
    sshs, ssgs = (ssh0, ssh1, ssh2), (ssg0, ssg1, ssg2)

    pltpu.sync_copy(zerosh.at[pl.ds(sid * RPT, RPT)], acch.at[pl.ds(sid * RPT, RPT)])
    pltpu.sync_copy(zerosg.at[pl.ds(sid * RPT, RPT)], accg.at[pl.ds(sid * RPT, RPT)])
    pltpu.sync_copy(rows2.at[pl.ds(wid * ITERS_S, ITERS_S)], idx2)
    plsc.subcore_barrier()

    def slot(jj, carry):
        for b in range(3):
            j = 3 * jj + b

            @pl.when(jnp.logical_and(j >= 3, j < ITERS_S))
            def _():
                pltpu.make_async_copy(bhs[b], acch.at[idx2.at[0]], sshs[b]).wait()
                pltpu.make_async_copy(bgs[b], accg.at[idx2.at[0]], ssgs[b]).wait()

            @pl.when(j < ITERS_S)
            def _():
                pltpu.async_copy(edath.at[pl.ds(base + j * CH_S, CH_S)], bhs[b],
                                 sihs[b])
                pltpu.async_copy(edatg.at[pl.ds(base + j * CH_S, CH_S)], bgs[b],
                                 sigs[b])

            jB = j - 1
            bB = (b - 1) % 3

            @pl.when(jnp.logical_and(jB >= 0, jB < ITERS_S))
            def _():
                pltpu.make_async_copy(edath.at[pl.ds(base, CH_S)], bhs[bB],
                                      sihs[bB]).wait()
                pltpu.make_async_copy(edatg.at[pl.ds(base, CH_S)], bgs[bB],
                                      sigs[bB]).wait()
                pltpu.async_copy(bhs[bB], acch.at[idx2.at[jB]], sshs[bB], add=True)
                pltpu.async_copy(bgs[bB], accg.at[idx2.at[jB]], ssgs[bB], add=True)
        return carry

    n_slots = ITERS_S + 1
    lax.fori_loop(0, (n_slots + 2) // 3, slot, 0)
    for k in range(3):
        b = (ITERS_S - 3 + k) % 3
        pltpu.make_async_copy(bhs[b], acch.at[idx2.at[0]], sshs[b]).wait()
        pltpu.make_async_copy(bgs[b], accg.at[idx2.at[0]], ssgs[b]).wait()
    plsc.subcore_barrier()
    pltpu.sync_copy(acch.at[pl.ds(sid * RPT, RPT)],
                    outh.at[cid].at[pl.ds(sid * RPT, RPT)])
    pltpu.sync_copy(accg.at[pl.ds(sid * RPT, RPT)],
                    outg.at[cid].at[pl.ds(sid * RPT, RPT)])


def _sc_scatter(edath, edatg, rows2, zerosh, zerosg):
    mesh = plsc.VectorSubcoreMesh(core_axis_name="c", subcore_axis_name="s")
    f = functools.partial(
        pl.kernel,
        out_type=[
            jax.ShapeDtypeStruct((NC, N_PAD, H), F32),
            jax.ShapeDtypeStruct((NC, N_PAD, GW), F32),
        ],
        mesh=mesh,
        compiler_params=pltpu.CompilerParams(use_tc_tiling_on_sc=False),
        scratch_types=[
            pltpu.VMEM((ITERS_S, CH_S), I32),
        ] + [pltpu.VMEM((CH_S, H), F32)] * 3
          + [pltpu.VMEM((CH_S, GW), F32)] * 3
          + [pltpu.VMEM_SHARED((N_PAD, H), F32),
             pltpu.VMEM_SHARED((N_PAD, GW), F32)]
          + [pltpu.SemaphoreType.DMA] * 12,
    )(_sc_scatter_body)
    return f(edath, edatg, rows2, zerosh, zerosg)


def kernel(h, x, edges, edge_attr, params):
    row = edges[0]
    col = edges[1]

    h_pad = jnp.pad(h, ((0, N_PAD - N), (0, 0)))
    cp = jnp.pad(x, ((0, N_PAD - N), (0, GW - 3)))
    ea = jnp.pad(edge_attr, ((0, E_PAD - E), (0, 0)))
    rowg = jnp.pad(row, (0, E_PAD - E))
    colg = jnp.pad(col, (0, E_PAD - E))
    eidx = jnp.stack([rowg, colg])
    rows_sc = jnp.pad(row, (0, E_PAD - E), constant_values=N)
    rows_sc2 = rows_sc.reshape(NW * ITERS_S, CH_S)
    zerosh = jnp.zeros((N_PAD, H), F32)
    zerosg = jnp.zeros((N_PAD, GW), F32)

    def lw(i):
        p = params["layers"][i]
        w1 = p["e1"]["W"]
        return {
            "w1r": w1[:, :H].T, "w1c": w1[:, H:2 * H].T,
            "wr": w1[:, 2 * H].reshape(1, H), "w1ea": w1[:, 2 * H + 1:].T,
            "b1": p["e1"]["b"].reshape(1, H),
            "w2": p["e2"]["W"].T, "b2": p["e2"]["b"].reshape(1, H),
            "wc1": p["c1"]["W"].T, "bc1": p["c1"]["b"].reshape(1, H),
            "wc2": p["c2"]["W"].reshape(1, H),
            "wn1h": p["n1"]["W"][:, :H].T, "wn1m": p["n1"]["W"][:, H:].T,
            "bn1": p["n1"]["b"].reshape(1, H),
            "wn2": p["n2"]["W"].T, "bn2": p["n2"]["b"].reshape(1, H),
        }

    lws = [lw(i) for i in range(3)]
    wemb_in = params["emb_in"]["W"].T
    bemb_in = params["emb_in"]["b"].reshape(1, H)
    wemb_out = params["emb_out"]["W"].T
    bemb_out = params["emb_out"]["b"].reshape(1, H)

    hcur, tp, tg = _tc_init(h_pad, cp, wemb_in, bemb_in,
                            lws[0]["w1r"], lws[0]["w1c"])
    for i in range(3):
        w = lws[i]
        sh, sg = _sc_gather(tp, tg, eidx)
        edath, edatg = _tc_edge(sh, sg, ea, w["w1ea"], w["b1"], w["wr"],
                                w["w2"], w["b2"], w["wc1"], w["bc1"], w["wc2"])
        ph, pg = _sc_scatter(edath, edatg, rows_sc2, zerosh, zerosg)
        last = i == 2
        if last:
            nw1r, nw1c = wemb_out, bemb_out
        else:
            nw1r, nw1c = lws[i + 1]["w1r"], lws[i + 1]["w1c"]
        res = _tc_node(hcur, cp, ph, pg,
                       w["wn1h"], w["wn1m"], w["bn1"], w["wn2"], w["bn2"],
                       nw1r, nw1c, last)
        if last:
            hcur, cp = res
        else:
            hcur, cp, tp, tg = res

    return hcur[:N], cp[:N, :3]


# speedup vs baseline: 4.3138x; 1.0336x over previous
import functools

import jax
import jax.numpy as jnp
from jax import lax
from jax.experimental import pallas as pl
from jax.experimental.pallas import tpu as pltpu
from jax.experimental.pallas import tpu_sc as plsc

F32 = jnp.float32
I32 = jnp.int32

N = 10000
E = 320000
H = 128
DE = 16
GW = 16
EPSV = 1e-8

NC, NS = 2, 16
NW = NC * NS
N_PAD = 10016
E_PAD = E
PER_T = E_PAD // NS
CH2 = 40
ITERS2 = PER_T // CH2
PER_W = E_PAD // NW
CH_S = 40
ITERS_S = PER_W // CH_S
RPT = N_PAD // NS

BE = 4000
BN = 2504


def _silu(v):
    return v * jax.nn.sigmoid(v)


def _init_body(h_ref, cp_ref, wemb_ref, bemb_ref, w1r_ref, w1c_ref,
               h0_ref, tp_ref, tg_ref):
    h0 = jnp.dot(h_ref[...], wemb_ref[...], preferred_element_type=F32) + bemb_ref[...]
    h0_ref[...] = h0
    cp = cp_ref[...]
    tp_ref[0] = jnp.dot(h0, w1r_ref[...], preferred_element_type=F32)
    tp_ref[1] = jnp.dot(h0, w1c_ref[...], preferred_element_type=F32)
    tg_ref[0] = cp
    tg_ref[1] = -cp


def _edge_body(sh0_ref, sh1_ref, sg0_ref, sg1_ref, ea_ref, w1ea_ref, b1_ref,
               wr_ref, w2_ref, b2_ref, wc1_ref, bc1_ref, wc2_ref,
               outh_ref, outg_ref):
    sh = sh0_ref[0] + sh1_ref[0]
    diff = sg0_ref[0] + sg1_ref[0]
    radial = jnp.sum(diff * diff, axis=1, keepdims=True)
    norm = jnp.sqrt(radial) + EPSV
    unit = diff / norm
    e1 = sh + radial * wr_ref[...] + b1_ref[...]
    e1 = e1 + jnp.dot(ea_ref[...], w1ea_ref[...], preferred_element_type=F32)
    m = _silu(e1)
    m = _silu(jnp.dot(m, w2_ref[...], preferred_element_type=F32) + b2_ref[...])
    ch = _silu(jnp.dot(m, wc1_ref[...], preferred_element_type=F32) + bc1_ref[...])
    c = jnp.sum(ch * wc2_ref[...], axis=1, keepdims=True)
    trans = unit * c
    lane = lax.broadcasted_iota(I32, trans.shape, 1)
    trans = jnp.where(lane == 3, 1.0, trans)
    outh_ref[...] = m
    outg_ref[...] = trans


def _node_body(h_ref, cp_ref, ph0_ref, ph1_ref, pg0_ref, pg1_ref,
               wn1h_ref, wn1m_ref, bn1_ref, wn2_ref, bn2_ref,
               w1r_ref, w1c_ref, h1_ref, cp1_ref, *rest, last):
    magg = ph0_ref[0] + ph1_ref[0]
    tail = pg0_ref[0] + pg1_ref[0]
    cnt = jnp.maximum(tail[:, 3:4], 1.0)
    lane = lax.broadcasted_iota(I32, tail.shape, 1)
    aggt = jnp.where(lane < 3, tail, 0.0)
    cp1 = cp_ref[...] + aggt / cnt
    cp1_ref[...] = cp1
    h = h_ref[...]
    o = _silu(jnp.dot(h, wn1h_ref[...], preferred_element_type=F32)
              + jnp.dot(magg, wn1m_ref[...], preferred_element_type=F32)
              + bn1_ref[...])
    o = jnp.dot(o, wn2_ref[...], preferred_element_type=F32) + bn2_ref[...]
    h1 = h + o
    if last:
        h1_ref[...] = jnp.dot(h1, w1r_ref[...], preferred_element_type=F32) + w1c_ref[...]
    else:
        tp_ref, tg_ref = rest
        h1_ref[...] = h1
        tp_ref[0] = jnp.dot(h1, w1r_ref[...], preferred_element_type=F32)
        tp_ref[1] = jnp.dot(h1, w1c_ref[...], preferred_element_type=F32)
        tg_ref[0] = cp1
        tg_ref[1] = -cp1


def _wspec(r, c):
    return pl.BlockSpec((r, c), lambda i: (0, 0))


def _tc_init(h_pad, cp, wemb, bemb, w1r, w1c):
    grid = (N_PAD // BN,)
    return pl.pallas_call(
        _init_body,
        grid=grid,
        in_specs=[
            pl.BlockSpec((BN, H), lambda i: (i, 0)),
            pl.BlockSpec((BN, GW), lambda i: (i, 0)),
            _wspec(H, H), _wspec(1, H), _wspec(H, H), _wspec(H, H),
        ],
        out_specs=[
            pl.BlockSpec((BN, H), lambda i: (i, 0)),
            pl.BlockSpec((2, BN, H), lambda i: (0, i, 0)),
            pl.BlockSpec((2, BN, GW), lambda i: (0, i, 0)),
        ],
        out_shape=[
            jax.ShapeDtypeStruct((N_PAD, H), F32),
            jax.ShapeDtypeStruct((2, N_PAD, H), F32),
            jax.ShapeDtypeStruct((2, N_PAD, GW), F32),
        ],
    )(h_pad, cp, wemb, bemb, w1r, w1c)


def _tc_edge(sh, sg, ea, w1ea, b1, wr, w2, b2, wc1, bc1, wc2):
    grid = (E_PAD // BE,)
    return pl.pallas_call(
        _edge_body,
        grid=grid,
        in_specs=[
            pl.BlockSpec((1, BE, H), lambda i: (0, i, 0)),
            pl.BlockSpec((1, BE, H), lambda i: (1, i, 0)),
            pl.BlockSpec((1, BE, GW), lambda i: (0, i, 0)),
            pl.BlockSpec((1, BE, GW), lambda i: (1, i, 0)),
            pl.BlockSpec((BE, DE), lambda i: (i, 0)),
            _wspec(DE, H), _wspec(1, H), _wspec(1, H),
            _wspec(H, H), _wspec(1, H),
            _wspec(H, H), _wspec(1, H), _wspec(1, H),
        ],
        out_specs=[
            pl.BlockSpec((BE, H), lambda i: (i, 0)),
            pl.BlockSpec((BE, GW), lambda i: (i, 0)),
        ],
        out_shape=[
            jax.ShapeDtypeStruct((E_PAD, H), F32),
            jax.ShapeDtypeStruct((E_PAD, GW), F32),
        ],
    )(sh, sh, sg, sg, ea, w1ea, b1, wr, w2, b2, wc1, bc1, wc2)


def _tc_node(h, cp, ph, pg, wn1h, wn1m, bn1, wn2, bn2, w1r, w1c, last):
    grid = (N_PAD // BN,)
    out_specs = [
        pl.BlockSpec((BN, H), lambda i: (i, 0)),
        pl.BlockSpec((BN, GW), lambda i: (i, 0)),
    ]
    out_shape = [
        jax.ShapeDtypeStruct((N_PAD, H), F32),
        jax.ShapeDtypeStruct((N_PAD, GW), F32),
    ]
    if not last:
        out_specs += [
            pl.BlockSpec((2, BN, H), lambda i: (0, i, 0)),
            pl.BlockSpec((2, BN, GW), lambda i: (0, i, 0)),
        ]
        out_shape += [
            jax.ShapeDtypeStruct((2, N_PAD, H), F32),
            jax.ShapeDtypeStruct((2, N_PAD, GW), F32),
        ]
    return pl.pallas_call(
        functools.partial(_node_body, last=last),
        grid=grid,
        in_specs=[
            pl.BlockSpec((BN, H), lambda i: (i, 0)),
            pl.BlockSpec((BN, GW), lambda i: (i, 0)),
            pl.BlockSpec((1, BN, H), lambda i: (0, i, 0)),
            pl.BlockSpec((1, BN, H), lambda i: (1, i, 0)),
            pl.BlockSpec((1, BN, GW), lambda i: (0, i, 0)),
            pl.BlockSpec((1, BN, GW), lambda i: (1, i, 0)),
            _wspec(H, H), _wspec(H, H), _wspec(1, H),
            _wspec(H, H), _wspec(1, H), _wspec(H, H),
            _wspec(H, H) if not last else _wspec(1, H),
        ],
        out_specs=out_specs,
        out_shape=out_shape,
    )(h, cp, ph, ph, pg, pg, wn1h, wn1m, bn1, wn2, bn2, w1r, w1c)


def _sc_gather_body(tp, tg, eidx, outh, outg, idx_v,
                    bp0, bp1, bg0, bg1, tblp, tblg,
                    sp0, sp1, sg0, sg1, sop0, sop1, sog0, sog1):
    cid = lax.axis_index("c")
    sid = lax.axis_index("s")
    tbase = sid * PER_T
    bps, bgs = (bp0, bp1), (bg0, bg1)
    sps, sgs = (sp0, sp1), (sg0, sg1)
    sops, sogs = (sop0, sop1), (sog0, sog1)

    pltpu.sync_copy(tp.at[cid].at[pl.ds(sid * RPT, RPT)],
                    tblp.at[pl.ds(sid * RPT, RPT)])
    pltpu.sync_copy(tg.at[cid].at[pl.ds(sid * RPT, RPT)],
                    tblg.at[pl.ds(sid * RPT, RPT)])
    pltpu.sync_copy(eidx.at[cid].at[pl.ds(tbase, PER_T)], idx_v)
    plsc.subcore_barrier()

    def slot(jj, carry):
        for b in (0, 1):
            j = 2 * jj + b

            @pl.when(jnp.logical_and(j >= 2, j < ITERS2))
            def _():
                pltpu.make_async_copy(bps[b], outh.at[0].at[pl.ds(0, CH2)],
                                      sops[b]).wait()
                pltpu.make_async_copy(bgs[b], outg.at[0].at[pl.ds(0, CH2)],
                                      sogs[b]).wait()

            @pl.when(j < ITERS2)
            def _():
                pltpu.async_copy(tblp.at[idx_v.at[pl.ds(j * CH2, CH2)]], bps[b],
                                 sps[b])
                pltpu.async_copy(tblg.at[idx_v.at[pl.ds(j * CH2, CH2)]], bgs[b],
                                 sgs[b])

            jB = j - 1
            bB = 1 - b

            @pl.when(jnp.logical_and(jB >= 0, jB < ITERS2))
            def _():
                pltpu.make_async_copy(tblp.at[idx_v.at[pl.ds(0, CH2)]], bps[bB],
                                      sps[bB]).wait()
                pltpu.make_async_copy(tblg.at[idx_v.at[pl.ds(0, CH2)]], bgs[bB],
                                      sgs[bB]).wait()
                pltpu.async_copy(bps[bB], outh.at[cid].at[pl.ds(tbase + jB * CH2, CH2)],
                                 sops[bB])
                pltpu.async_copy(bgs[bB], outg.at[cid].at[pl.ds(tbase + jB * CH2, CH2)],
                                 sogs[bB])
        return carry

    n_slots = ITERS2 + 1
    lax.fori_loop(0, (n_slots + 1) // 2, slot, 0)
    for b in (0, 1):
        pltpu.make_async_copy(bps[b], outh.at[0].at[pl.ds(0, CH2)], sops[b]).wait()
        pltpu.make_async_copy(bgs[b], outg.at[0].at[pl.ds(0, CH2)], sogs[b]).wait()


def _sc_gather(tp, tg, eidx):
    mesh = plsc.VectorSubcoreMesh(core_axis_name="c", subcore_axis_name="s")
    f = functools.partial(
        pl.kernel,
        out_type=[
            jax.ShapeDtypeStruct((NC, E_PAD, H), F32),
            jax.ShapeDtypeStruct((NC, E_PAD, GW), F32),
        ],
        mesh=mesh,
        compiler_params=pltpu.CompilerParams(use_tc_tiling_on_sc=False),
        scratch_types=[
            pltpu.VMEM((PER_T,), I32),
            pltpu.VMEM((CH2, H), F32),
            pltpu.VMEM((CH2, H), F32),
            pltpu.VMEM((CH2, GW), F32),
            pltpu.VMEM((CH2, GW), F32),
            pltpu.VMEM_SHARED((N_PAD, H), F32),
            pltpu.VMEM_SHARED((N_PAD, GW), F32),
        ] + [pltpu.SemaphoreType.DMA] * 8,
    )(_sc_gather_body)
    return f(tp, tg, eidx)


def _sc_scatter_body(edath, edatg, rows2, zerosh, zerosg, outh, outg, idx2,
                     bh0, bh1, bh2, bg0, bg1, bg2, acch, accg,
                     sih0, sih1, sih2, sig0, sig1, sig2,
                     ssh0, ssh1, ssh2, ssg0, ssg1, ssg2):
    cid = lax.axis_index("c")
    sid = lax.axis_index("s")
    wid = sid * NC + cid
    base = wid * PER_W
    bhs, bgs = (bh0, bh1, bh2), (bg0, bg1, bg2)
    sihs, sigs = (sih0, sih1, sih2), (sig0, sig1, sig2)
    sshs, ssgs = (ssh0, ssh1, ssh2), (ssg0, ssg1, ssg2)

    pltpu.sync_copy(zerosh.at[pl.ds(sid * RPT, RPT)], acch.at[pl.ds(sid * RPT, RPT)])
    pltpu.sync_copy(zerosg.at[pl.ds(sid * RPT, RPT)], accg.at[pl.ds(sid * RPT, RPT)])
    pltpu.sync_copy(rows2.at[pl.ds(wid * ITERS_S, ITERS_S)], idx2)
    plsc.subcore_barrier()

    def slot(jj, carry):
        for b in range(3):
            j = 3 * jj + b

            @pl.when(jnp.logical_and(j >= 3, j < ITERS_S))
            def _():
                pltpu.make_async_copy(bhs[b], acch.at[idx2.at[0]], sshs[b]).wait()
                pltpu.make_async_copy(bgs[b], accg.at[idx2.at[0]], ssgs[b]).wait()

            @pl.when(j < ITERS_S)
            def _():
                pltpu.async_copy(edath.at[pl.ds(base + j * CH_S, CH_S)], bhs[b],
                                 sihs[b])
                pltpu.async_copy(edatg.at[pl.ds(base + j * CH_S, CH_S)], bgs[b],
                                 sigs[b])

            jB = j - 1
            bB = (b - 1) % 3

            @pl.when(jnp.logical_and(jB >= 0, jB < ITERS_S))
            def _():
                pltpu.make_async_copy(edath.at[pl.ds(base, CH_S)], bhs[bB],
                                      sihs[bB]).wait()
                pltpu.make_async_copy(edatg.at[pl.ds(base, CH_S)], bgs[bB],
                                      sigs[bB]).wait()
                pltpu.async_copy(bhs[bB], acch.at[idx2.at[jB]], sshs[bB], add=True)
                pltpu.async_copy(bgs[bB], accg.at[idx2.at[jB]], ssgs[bB], add=True)
        return carry

    n_slots = ITERS_S + 1
    lax.fori_loop(0, (n_slots + 2) // 3, slot, 0)
    for k in range(3):
        b = (ITERS_S - 3 + k) % 3
        pltpu.make_async_copy(bhs[b], acch.at[idx2.at[0]], sshs[b]).wait()
        pltpu.make_async_copy(bgs[b], accg.at[idx2.at[0]], ssgs[b]).wait()
    plsc.subcore_barrier()
    pltpu.sync_copy(acch.at[pl.ds(sid * RPT, RPT)],
                    outh.at[cid].at[pl.ds(sid * RPT, RPT)])
    pltpu.sync_copy(accg.at[pl.ds(sid * RPT, RPT)],
                    outg.at[cid].at[pl.ds(sid * RPT, RPT)])


def _sc_scatter(edath, edatg, rows2, zerosh, zerosg):
    mesh = plsc.VectorSubcoreMesh(core_axis_name="c", subcore_axis_name="s")
    f = functools.partial(
        pl.kernel,
        out_type=[
            jax.ShapeDtypeStruct((NC, N_PAD, H), F32),
            jax.ShapeDtypeStruct((NC, N_PAD, GW), F32),
        ],
        mesh=mesh,
        compiler_params=pltpu.CompilerParams(use_tc_tiling_on_sc=False),
        scratch_types=[
            pltpu.VMEM((ITERS_S, CH_S), I32),
        ] + [pltpu.VMEM((CH_S, H), F32)] * 3
          + [pltpu.VMEM((CH_S, GW), F32)] * 3
          + [pltpu.VMEM_SHARED((N_PAD, H), F32),
             pltpu.VMEM_SHARED((N_PAD, GW), F32)]
          + [pltpu.SemaphoreType.DMA] * 12,
    )(_sc_scatter_body)
    return f(edath, edatg, rows2, zerosh, zerosg)


def kernel(h, x, edges, edge_attr, params):
    row = edges[0]
    col = edges[1]

    h_pad = jnp.pad(h, ((0, N_PAD - N), (0, 0)))
    cp = jnp.pad(x, ((0, N_PAD - N), (0, GW - 3)))
    ea = edge_attr
    eidx = jnp.stack([row, col])
    rows_sc2 = row.reshape(NW * ITERS_S, CH_S)
    zerosh = jnp.zeros((N_PAD, H), F32)
    zerosg = jnp.zeros((N_PAD, GW), F32)

    def lw(i):
        p = params["layers"][i]
        w1 = p["e1"]["W"]
        return {
            "w1r": w1[:, :H].T, "w1c": w1[:, H:2 * H].T,
            "wr": w1[:, 2 * H].reshape(1, H), "w1ea": w1[:, 2 * H + 1:].T,
            "b1": p["e1"]["b"].reshape(1, H),
            "w2": p["e2"]["W"].T, "b2": p["e2"]["b"].reshape(1, H),
            "wc1": p["c1"]["W"].T, "bc1": p["c1"]["b"].reshape(1, H),
            "wc2": p["c2"]["W"].reshape(1, H),
            "wn1h": p["n1"]["W"][:, :H].T, "wn1m": p["n1"]["W"][:, H:].T,
            "bn1": p["n1"]["b"].reshape(1, H),
            "wn2": p["n2"]["W"].T, "bn2": p["n2"]["b"].reshape(1, H),
        }

    lws = [lw(i) for i in range(3)]
    wemb_in = params["emb_in"]["W"].T
    bemb_in = params["emb_in"]["b"].reshape(1, H)
    wemb_out = params["emb_out"]["W"].T
    bemb_out = params["emb_out"]["b"].reshape(1, H)

    hcur, tp, tg = _tc_init(h_pad, cp, wemb_in, bemb_in,
                            lws[0]["w1r"], lws[0]["w1c"])
    for i in range(3):
        w = lws[i]
        sh, sg = _sc_gather(tp, tg, eidx)
        edath, edatg = _tc_edge(sh, sg, ea, w["w1ea"], w["b1"], w["wr"],
                                w["w2"], w["b2"], w["wc1"], w["bc1"], w["wc2"])
        ph, pg = _sc_scatter(edath, edatg, rows_sc2, zerosh, zerosg)
        last = i == 2
        if last:
            nw1r, nw1c = wemb_out, bemb_out
        else:
            nw1r, nw1c = lws[i + 1]["w1r"], lws[i + 1]["w1c"]
        res = _tc_node(hcur, cp, ph, pg,
                       w["wn1h"], w["wn1m"], w["bn1"], w["wn2"], w["bn2"],
                       nw1r, nw1c, last)
        if last:
            hcur, cp = res
        else:
            hcur, cp, tp, tg = res

    return hcur[:N], cp[:N, :3]
